# baseline (speedup 1.0000x reference)
import functools

import jax
import jax.numpy as jnp
from jax import lax
from jax.experimental import pallas as pl
from jax.experimental.pallas import tpu as pltpu
from jax.experimental.pallas import tpu_sc as plsc

N_USERS = 1000000
N_MOVIES = 100000
N_FACTORS = 64
BATCH = 16384
NC = 2
NS = 16
NW = NC * NS
BPW = BATCH // NW
LANES = 16
NGROUP = BPW // LANES
NROUND = 2
DPR = N_FACTORS // NROUND
RLEN = DPR * BPW


def _cf_body(uids, mids, utab, mtab, ubtab, mbtab, gbias, out,
             uidx_v, midx_v, ubig_v, mbig_v, udat_v, mdat_v,
             ub_v, mb_v, gb_v, acc_v, out_v, sem):
    wid = lax.axis_index("s") * NC + lax.axis_index("c")
    base = wid * BPW

    pltpu.sync_copy(uids.at[pl.ds(base, BPW)], uidx_v)
    pltpu.sync_copy(mids.at[pl.ds(base, BPW)], midx_v)
    pltpu.sync_copy(gbias, gb_v.at[pl.ds(0, 1)])

    bias_copies = [
        pltpu.async_copy(ubtab.at[uidx_v], ub_v, sem),
        pltpu.async_copy(mbtab.at[midx_v], mb_v, sem),
    ]

    def zero_acc(g, c):
        acc_v[pl.ds(g * LANES, LANES)] = jnp.zeros((LANES,), jnp.float32)
        return c
    lax.fori_loop(0, NGROUP, zero_acc, 0)

    def build_idx(r):
        def body(i, c):
            dd = i // NGROUP
            g = i % NGROUP
            sl16 = pl.ds(g * LANES, LANES)
            dsl = pl.ds(dd * BPW + g * LANES, LANES)
            d = r * DPR + dd
            ubig_v[dsl] = uidx_v[sl16] + d * N_USERS
            mbig_v[dsl] = midx_v[sl16] + d * N_MOVIES
            return c
        lax.fori_loop(0, DPR * NGROUP, body, 0, unroll=4)

    def accumulate(r):
        def body(g, c):
            sl16 = pl.ds(g * LANES, LANES)
            def dot_body(dd, a):
                dsl = pl.ds(dd * BPW + g * LANES, LANES)
                return a + udat_v[dsl] * mdat_v[dsl]
            acc_v[sl16] = lax.fori_loop(0, DPR, dot_body, acc_v[sl16],
                                        unroll=8)
            return c
        lax.fori_loop(0, NGROUP, body, 0)

    for r in range(NROUND):
        build_idx(r)
        ucp = pltpu.async_copy(utab.at[ubig_v], udat_v, sem)
        mcp = pltpu.async_copy(mtab.at[mbig_v], mdat_v, sem)
        ucp.wait()
        mcp.wait()
        accumulate(r)

    for c in bias_copies:
        c.wait()

    gb_vec = jnp.zeros((LANES,), jnp.float32) + gb_v[...][0]

    def finish(g, carry):
        sl = pl.ds(g * LANES, LANES)
        rr = acc_v[sl] + ub_v[sl] + mb_v[sl] + gb_vec
        out_v[sl] = 1.0 / (1.0 + jnp.exp(-rr))
        return carry

    lax.fori_loop(0, NGROUP, finish, 0)
    pltpu.sync_copy(out_v, out.at[pl.ds(base, BPW)])


@jax.jit
def _cf_call(uids, mids, utab, mtab, ubtab, mbtab, gbias):
    mesh = plsc.VectorSubcoreMesh(core_axis_name="c", subcore_axis_name="s")
    return pl.kernel(
        _cf_body,
        out_type=jax.ShapeDtypeStruct((BATCH,), jnp.float32),
        mesh=mesh,
        scratch_types=[
            pltpu.VMEM((BPW,), jnp.int32),
            pltpu.VMEM((BPW,), jnp.int32),
            pltpu.VMEM((RLEN,), jnp.int32),
            pltpu.VMEM((RLEN,), jnp.int32),
            pltpu.VMEM((RLEN,), jnp.float32),
            pltpu.VMEM((RLEN,), jnp.float32),
            pltpu.VMEM((BPW,), jnp.float32),
            pltpu.VMEM((BPW,), jnp.float32),
            pltpu.VMEM((LANES,), jnp.float32),
            pltpu.VMEM((BPW,), jnp.float32),
            pltpu.VMEM((BPW,), jnp.float32),
            pltpu.SemaphoreType.DMA,
        ],
    )(uids, mids, utab, mtab, ubtab, mbtab, gbias)


def kernel(user_ids, movie_ids, user_table, movie_table, user_bias_table,
           movie_bias_table, global_bias):
    return _cf_call(user_ids.astype(jnp.int32), movie_ids.astype(jnp.int32),
                    user_table.T.reshape(-1), movie_table.T.reshape(-1),
                    user_bias_table.reshape(-1),
                    movie_bias_table.reshape(-1), global_bias)
